# baseline (speedup 1.0000x reference)
import functools
import jax
import jax.numpy as jnp
import numpy as np
from jax import lax
from jax.experimental import pallas as pl
from jax.experimental.pallas import tpu as pltpu
from jax.experimental.pallas import tpu_sc as plsc

B = 512; K = 24; P = 16; EP = 32; E = 16384
FIN = 1; HID = 64; HEADS = 2; OUT = 32; NH = 128; ORIG = 256; NCLS = 2
BN_SCALE = 1.0 / np.sqrt(1.0 + 1e-5)
LN_EPS = 1e-5
BB = 64

_pc = pl.pallas_call


def _leaky(x):
    return jnp.where(x >= 0, x, 0.2 * x)


def _plane_body(idx_ref, x_ref, xf_ref, orig_ref, Wg_ref, b1_ref,
                wA_ref, wB_ref, W2_ref, res2_ref, b2_ref,
                D1_ref, d1b_ref, D2f_ref, d2b_ref, Wfo_ref, Wfg_ref, bf_ref,
                hN_ref, loss_ref):
    src = idx_ref[0:1, 0:EP]
    dst = idx_ref[1:2, 0:EP]
    pio = jax.lax.broadcasted_iota(jnp.int32, (P, EP), 0).astype(jnp.float32)
    Gs = (pio == src).astype(jnp.float32)
    Gd = (pio == dst).astype(jnp.float32)
    src2 = idx_ref[0:1, :]
    dst2 = idx_ref[1:2, :]
    hrow = idx_ref[2:3, :]
    rio = jax.lax.broadcasted_iota(jnp.int32, (2 * P, 2 * EP), 0)
    rp = jnp.remainder(rio, P).astype(jnp.float32)
    rh = (rio // P).astype(jnp.float32)
    Gs2 = (rp == src2).astype(jnp.float32) * (rh == hrow)
    Gd2 = (rp == dst2).astype(jnp.float32) * (rh == hrow)

    x = x_ref[...]
    b1 = b1_ref[...]
    b2 = b2_ref[...]
    d1b = d1b_ref[...]
    d2b = d2b_ref[...]
    xs2 = jax.lax.dot_general(x, Gs2[0:P] + Gs2[P:2 * P],
                              (((2,), (0,)), ((), ())))
    xd2 = jax.lax.dot_general(x, Gd2[0:P] + Gd2[P:2 * P],
                              (((2,), (0,)), ((), ())))
    wA = wA_ref[...]
    wB = wB_ref[...]
    e12 = _leaky(xs2 * wA[:, None, :] + xd2 * wB[:, None, :])
    m = jnp.max(e12, axis=2, keepdims=True)
    a12 = jnp.exp(e12 - m)
    den12 = jax.lax.dot_general(a12, Gd2, (((2,), (1,)), ((), ())))
    num12 = jax.lax.dot_general(a12 * xs2, Gd2, (((2,), (1,)), ((), ())))
    s1p = jnp.where(den12 > 0, num12 / jnp.maximum(den12, 1e-30), 0.0)
    s1_0 = s1p[:, :, 0:P]
    s1_1 = s1p[:, :, P:2 * P]
    G = jnp.concatenate([s1_0[..., None], s1_1[..., None], x[..., None]],
                        axis=-1).reshape(K, BB * P, 3)
    h1r = jnp.maximum(
        BN_SCALE * (jax.lax.dot_general(G, Wg_ref[...],
                                        (((2,), (1,)), ((0,), (0,))))
                    + b1[:, None, :]), 0.0)
    h1 = h1r.reshape(K, BB, P, HEADS * HID)

    feat2e = jax.lax.dot_general(h1r, W2_ref[...],
                                 (((2,), (1,)), ((0,), (0,))))
    feat2 = feat2e[:, :, 0:OUT]
    el2 = feat2e[:, :, OUT].reshape(K, BB, P)
    er2 = feat2e[:, :, OUT + 1].reshape(K, BB, P)
    e2 = _leaky(jax.lax.dot_general(el2, Gs, (((2,), (0,)), ((), ())))
                + jax.lax.dot_general(er2, Gd, (((2,), (0,)), ((), ()))))
    m2 = jnp.max(e2, axis=2, keepdims=True)
    a2 = jnp.exp(e2 - m2)
    den2 = jax.lax.dot_general(a2, Gd, (((2,), (1,)), ((), ())))
    A1 = (Gd[None, None] * a2[:, :, None, :]).reshape(K * BB, P, EP)
    M2 = jax.lax.dot_general(A1, Gs, (((2,), (1,)), ((), ())))
    feat2b = feat2.reshape(K * BB, P, OUT)
    num2 = jax.lax.dot_general(M2, feat2b,
                               (((2,), (1,)), ((0,), (0,))))
    inv2 = jnp.where(den2 > 0, 1.0 / jnp.maximum(den2, 1e-30), 0.0)
    out2 = num2 * inv2.reshape(K * BB, P, 1)
    res2o = jax.lax.dot_general(h1r, res2_ref[...],
                                (((2,), (1,)), ((0,), (0,))))
    h2 = jnp.maximum(
        BN_SCALE * ((out2.reshape(K, BB, P, OUT)
                     + res2o.reshape(K, BB, P, OUT))
                    + b2[:, None, None, :]), 0.0)

    graph_rep = jnp.mean(h2, axis=2)

    h2r = h2.reshape(K, BB * P, OUT)
    d1 = jnp.maximum(
        BN_SCALE * (jax.lax.dot_general(h2r, D1_ref[...],
                                        (((2,), (1,)), ((0,), (0,))))
                    + d1b[:, None, :]), 0.0)
    rec = jax.lax.dot_general(d1, D2f_ref[...],
                              (((2,), (1,)), ((0,), (0,)))) + d2b[:, 0:1]
    xr = xf_ref[...]
    sq = (rec - xr) ** 2
    partial = jnp.sum(sq, axis=0, keepdims=True) * (1.0 / (P * B * K))

    @pl.when(pl.program_id(0) == 0)
    def _():
        loss_ref[...] = jnp.zeros((1, BB * P), jnp.float32)
    loss_ref[...] += partial

    g = jax.lax.dot_general(graph_rep, Wfg_ref[...],
                            (((2,), (1,)), ((0,), (0,))))
    gsum = jnp.sum(g, axis=0)
    ho = jnp.dot(orig_ref[...], Wfo_ref[...])
    hN_ref[...] = jnp.maximum(BN_SCALE * (gsum + ho + bf_ref[...][None, :]),
                              0.0)


def _count_body(ps_hbm, pd_hbm, z_hbm, out_hbm, ps_v, pd_v, flat_v, ones_v,
                c_sh):
    c = lax.axis_index("c")
    s = lax.axis_index("s")
    wid = s * 2 + c
    seg = 262144 // 16
    pltpu.sync_copy(z_hbm, c_sh.at[pl.ds(s * seg, seg)])
    plsc.subcore_barrier()
    base = wid * (E // 32)
    pltpu.sync_copy(ps_hbm.at[pl.ds(base, E // 32)], ps_v)
    pltpu.sync_copy(pd_hbm.at[pl.ds(base, E // 32)], pd_v)
    for j in range(8):
        ones_v[pl.ds(j * 16, 16)] = jnp.ones((16,), jnp.float32)
    for j in range(32):
        s16 = ps_v[pl.ds(j * 16, 16)]
        d16 = pd_v[pl.ds(j * 16, 16)]
        flat_v[j // 8, pl.ds((j % 8) * 16, 16)] = d16 * B + s16
    for r in range(4):
        pltpu.sync_copy(ones_v, c_sh.at[flat_v.at[r]], add=True)
    plsc.subcore_barrier()
    pltpu.sync_copy(c_sh.at[pl.ds(s * seg, seg)],
                    out_hbm.at[c, pl.ds(s * seg, seg)])


def _build_counts(psrc, pdst):
    mesh = plsc.VectorSubcoreMesh(core_axis_name="c", subcore_axis_name="s")
    k = functools.partial(
        pl.kernel, mesh=mesh,
        out_type=jax.ShapeDtypeStruct((2, B * B), jnp.float32),
        scratch_types=[
            pltpu.VMEM((E // 32,), jnp.int32),
            pltpu.VMEM((E // 32,), jnp.int32),
            pltpu.VMEM((4, 128), jnp.int32),
            pltpu.VMEM((128,), jnp.float32),
            pltpu.VMEM_SHARED((B * B,), jnp.float32),
        ],
    )(_count_body)
    return k(psrc, pdst, jnp.zeros((262144 // 16,), jnp.float32))


def _patient_body(cpair_ref, mrow_ref, mcol_ref, hN_ref,
                  Wc_ref, bc_ref, Wl1_ref, bl1_ref, Wl2_ref, bl2_ref,
                  out_ref):
    C = cpair_ref[0] + cpair_ref[1]

    degS = jnp.sum(C, axis=0, keepdims=True)
    degD = jnp.sum(C, axis=1, keepdims=True)
    ns = jax.lax.rsqrt(jnp.maximum(degS, 1.0))
    nd = jax.lax.rsqrt(jnp.maximum(degD, 1.0))
    A = C * mcol_ref[...] * (mrow_ref[...] * ns)

    hN = hN_ref[...]
    hcur = hN
    hsum = hN
    for i in range(3):
        agg = jnp.dot(A, hcur)
        rst = jnp.dot(agg * nd, Wc_ref[i]) + bc_ref[i, :][None, :]
        hcur = jnp.maximum(BN_SCALE * rst, 0.0) + hcur
        hsum = hsum + hcur
    hm = hsum * 0.25

    z = jnp.dot(hm, Wl1_ref[...]) + bl1_ref[...][None, :]
    mu = jnp.mean(z, axis=-1, keepdims=True)
    var = jnp.mean((z - mu) ** 2, axis=-1, keepdims=True)
    zn = jnp.maximum((z - mu) * jax.lax.rsqrt(var + LN_EPS), 0.0)
    out_ref[...] = jnp.dot(zn, Wl2_ref[...]) + bl2_ref[...][None, :]


def kernel(plane_feats, plane_edge_index, patient_edge_index,
           original_features, mask, W1, a1s, a1d, res1, b1, W2, a2s, a2d,
           res2, b2, D1, d1b, D2, d2b, Wf, bf, Wc, bc, Wl1, bl1, Wl2, bl2):
    f32 = jnp.float32
    xk = plane_feats.reshape(B, K, P).transpose(1, 0, 2)
    xflat = xk.reshape(K, B * P)
    ef = plane_edge_index.astype(f32)
    idx8 = jnp.zeros((8, 2 * EP), f32)
    idx8 = idx8.at[0:2].set(jnp.tile(ef, (1, 2)))
    idx8 = idx8.at[2, EP:].set(1.0)

    W1k = W1[:, 0, :]
    res1k = res1[:, 0, :]
    w1s = jnp.sum(W1k.reshape(K, HEADS, HID) * a1s, axis=-1)
    w1d = jnp.sum(W1k.reshape(K, HEADS, HID) * a1d, axis=-1)
    wA = jnp.repeat(w1s, EP, axis=1)
    wB = jnp.repeat(w1d, EP, axis=1)
    hsel = (jnp.arange(HEADS * HID) >= HID).astype(f32)
    Wg = jnp.stack([W1k * (1.0 - hsel), W1k * hsel, res1k], axis=1)
    w2s = jnp.einsum('kfo,ko->kf', W2, a2s[:, 0])
    w2d = jnp.einsum('kfo,ko->kf', W2, a2d[:, 0])
    W2e = jnp.concatenate([W2, w2s[:, :, None], w2d[:, :, None],
                           jnp.zeros((K, NH, 6), f32)], axis=2)
    D2f = D2[:, :, 0]
    d2b8 = jnp.broadcast_to(d2b, (K, 8)) if d2b.shape[1] == 1 else d2b
    Wfo = Wf[:ORIG]
    Wfg = Wf[ORIG:].reshape(K, OUT, NH)

    nblk = B // BB
    grid_spec = pl.GridSpec(
        grid=(nblk,),
        in_specs=[
            pl.BlockSpec((8, 2 * EP), lambda i: (0, 0)),
            pl.BlockSpec((K, BB, P), lambda i: (0, i, 0)),
            pl.BlockSpec((K, BB * P), lambda i: (0, i)),
            pl.BlockSpec((BB, ORIG), lambda i: (i, 0)),
            pl.BlockSpec((K, 3, NH), lambda i: (0, 0, 0)),
            pl.BlockSpec((K, NH), lambda i: (0, 0)),
            pl.BlockSpec((K, 2 * EP), lambda i: (0, 0)),
            pl.BlockSpec((K, 2 * EP), lambda i: (0, 0)),
            pl.BlockSpec((K, NH, OUT + 8), lambda i: (0, 0, 0)),
            pl.BlockSpec((K, NH, OUT), lambda i: (0, 0, 0)),
            pl.BlockSpec((K, OUT), lambda i: (0, 0)),
            pl.BlockSpec((K, OUT, NH), lambda i: (0, 0, 0)),
            pl.BlockSpec((K, NH), lambda i: (0, 0)),
            pl.BlockSpec((K, NH), lambda i: (0, 0)),
            pl.BlockSpec((K, 8), lambda i: (0, 0)),
            pl.BlockSpec((ORIG, NH), lambda i: (0, 0)),
            pl.BlockSpec((K, OUT, NH), lambda i: (0, 0, 0)),
            pl.BlockSpec((NH,), lambda i: (0,)),
        ],
        out_specs=[
            pl.BlockSpec((BB, NH), lambda i: (i, 0)),
            pl.BlockSpec((1, BB * P), lambda i: (0, 0)),
        ],
    )
    hN, loss = _pc(
        _plane_body,
        grid_spec=grid_spec,
        out_shape=[jax.ShapeDtypeStruct((B, NH), f32),
                   jax.ShapeDtypeStruct((1, BB * P), f32)],
    )(idx8, xk, xflat, original_features, Wg, b1, wA, wB, W2e, res2, b2,
      D1, d1b, D2f, d2b8, Wfo, Wfg, bf)

    cpair = _build_counts(patient_edge_index[0], patient_edge_index[1])
    cpair = cpair.reshape(2, B, B)
    maskf = mask.astype(f32)
    mrow = maskf.reshape(1, B)
    mcol = maskf.reshape(B, 1)

    logits = _pc(
        _patient_body,
        out_shape=jax.ShapeDtypeStruct((B, NCLS), f32),
    )(cpair, mrow, mcol, hN, Wc, bc, Wl1, bl1, Wl2, bl2)

    return logits, jnp.sum(loss)


# speedup vs baseline: 1.0223x; 1.0223x over previous
import functools
import jax
import jax.numpy as jnp
import numpy as np
from jax import lax
from jax.experimental import pallas as pl
from jax.experimental.pallas import tpu as pltpu
from jax.experimental.pallas import tpu_sc as plsc

B = 512; K = 24; P = 16; EP = 32; E = 16384
FIN = 1; HID = 64; HEADS = 2; OUT = 32; NH = 128; ORIG = 256; NCLS = 2
BN_SCALE = 1.0 / np.sqrt(1.0 + 1e-5)
LN_EPS = 1e-5
BB = 64

_pc = pl.pallas_call


def _leaky(x):
    return jnp.where(x >= 0, x, 0.2 * x)


def _plane_body(idx_ref, x_ref, xf_ref, orig_ref, Wg_ref, b1_ref,
                wA_ref, wB_ref, w2s_ref, w2d_ref, W2_ref, b2_ref,
                D1_ref, d1b_ref, D2f_ref, d2b_ref, Wfo_ref, Wfg_ref, bf_ref,
                hN_ref, loss_ref):
    src = idx_ref[0:1, 0:EP]
    dst = idx_ref[1:2, 0:EP]
    pio = jax.lax.broadcasted_iota(jnp.int32, (P, EP), 0).astype(jnp.float32)
    Gs = (pio == src).astype(jnp.float32)
    Gd = (pio == dst).astype(jnp.float32)
    src2 = idx_ref[0:1, :]
    dst2 = idx_ref[1:2, :]
    hrow = idx_ref[2:3, :]
    rio = jax.lax.broadcasted_iota(jnp.int32, (2 * P, 2 * EP), 0)
    rp = jnp.remainder(rio, P).astype(jnp.float32)
    rh = (rio // P).astype(jnp.float32)
    Gs2 = (rp == src2).astype(jnp.float32) * (rh == hrow)
    Gd2 = (rp == dst2).astype(jnp.float32) * (rh == hrow)

    x = x_ref[...]
    b1 = b1_ref[...]
    b2 = b2_ref[...]
    d1b = d1b_ref[...]
    d2b = d2b_ref[...]
    xs2 = jax.lax.dot_general(x, Gs2[0:P] + Gs2[P:2 * P],
                              (((2,), (0,)), ((), ())))
    xd2 = jax.lax.dot_general(x, Gd2[0:P] + Gd2[P:2 * P],
                              (((2,), (0,)), ((), ())))
    wA = wA_ref[...]
    wB = wB_ref[...]
    e12 = _leaky(xs2 * wA[:, None, :] + xd2 * wB[:, None, :])
    m = jnp.max(e12, axis=2, keepdims=True)
    a12 = jnp.exp(e12 - m)
    den12 = jax.lax.dot_general(a12, Gd2, (((2,), (1,)), ((), ())))
    num12 = jax.lax.dot_general(a12 * xs2, Gd2, (((2,), (1,)), ((), ())))
    s1p = jnp.where(den12 > 0, num12 / jnp.maximum(den12, 1e-30), 0.0)
    s1_0 = s1p[:, :, 0:P]
    s1_1 = s1p[:, :, P:2 * P]
    G = jnp.concatenate([s1_0[..., None], s1_1[..., None], x[..., None]],
                        axis=-1).reshape(K, BB * P, 3)
    h1r = jnp.maximum(
        BN_SCALE * (jax.lax.dot_general(G, Wg_ref[...],
                                        (((2,), (1,)), ((0,), (0,))))
                    + b1[:, None, :]), 0.0)
    h1 = h1r.reshape(K, BB, P, HEADS * HID)

    feat2w = jax.lax.dot_general(h1r, W2_ref[...],
                                 (((2,), (1,)), ((0,), (0,))))
    feat2 = feat2w[:, :, 0:OUT]
    el2 = jnp.sum(h1 * w2s_ref[...][:, None, None, :], axis=-1)
    er2 = jnp.sum(h1 * w2d_ref[...][:, None, None, :], axis=-1)
    e2 = _leaky(jax.lax.dot_general(el2, Gs, (((2,), (0,)), ((), ())))
                + jax.lax.dot_general(er2, Gd, (((2,), (0,)), ((), ()))))
    m2 = jnp.max(e2, axis=2, keepdims=True)
    a2 = jnp.exp(e2 - m2)
    den2 = jax.lax.dot_general(a2, Gd, (((2,), (1,)), ((), ())))
    A1 = (Gd[None, None] * a2[:, :, None, :]).reshape(K * BB, P, EP)
    M2 = jax.lax.dot_general(A1, Gs, (((2,), (1,)), ((), ())))
    feat2b = feat2.reshape(K * BB, P, OUT)
    num2 = jax.lax.dot_general(M2, feat2b,
                               (((2,), (1,)), ((0,), (0,))))
    inv2 = jnp.where(den2 > 0, 1.0 / jnp.maximum(den2, 1e-30), 0.0)
    out2 = num2 * inv2.reshape(K * BB, P, 1)
    res2o = feat2w[:, :, OUT:2 * OUT]
    h2 = jnp.maximum(
        BN_SCALE * ((out2.reshape(K, BB, P, OUT)
                     + res2o.reshape(K, BB, P, OUT))
                    + b2[:, None, None, :]), 0.0)

    graph_rep = jnp.mean(h2, axis=2)

    h2r = h2.reshape(K, BB * P, OUT)
    d1 = jnp.maximum(
        BN_SCALE * (jax.lax.dot_general(h2r, D1_ref[...],
                                        (((2,), (1,)), ((0,), (0,))))
                    + d1b[:, None, :]), 0.0)
    rec = jax.lax.dot_general(d1, D2f_ref[...],
                              (((2,), (1,)), ((0,), (0,)))) + d2b[:, 0:1]
    xr = xf_ref[...]
    sq = (rec - xr) ** 2
    partial = jnp.sum(sq, axis=0, keepdims=True) * (1.0 / (P * B * K))

    @pl.when(pl.program_id(0) == 0)
    def _():
        loss_ref[...] = jnp.zeros((1, BB * P), jnp.float32)
    loss_ref[...] += partial

    g = jax.lax.dot_general(graph_rep, Wfg_ref[...],
                            (((2,), (1,)), ((0,), (0,))))
    gsum = jnp.sum(g, axis=0)
    ho = jnp.dot(orig_ref[...], Wfo_ref[...])
    hN_ref[...] = jnp.maximum(BN_SCALE * (gsum + ho + bf_ref[...][None, :]),
                              0.0)


def _count_body(ps_hbm, pd_hbm, z_hbm, out_hbm, ps_v, pd_v, flat_v, ones_v,
                c_sh):
    c = lax.axis_index("c")
    s = lax.axis_index("s")
    wid = s * 2 + c
    seg = 262144 // 16
    pltpu.sync_copy(z_hbm, c_sh.at[pl.ds(s * seg, seg)])
    plsc.subcore_barrier()
    base = wid * (E // 32)
    pltpu.sync_copy(ps_hbm.at[pl.ds(base, E // 32)], ps_v)
    pltpu.sync_copy(pd_hbm.at[pl.ds(base, E // 32)], pd_v)
    for j in range(8):
        ones_v[pl.ds(j * 16, 16)] = jnp.ones((16,), jnp.float32)
    for j in range(32):
        s16 = ps_v[pl.ds(j * 16, 16)]
        d16 = pd_v[pl.ds(j * 16, 16)]
        flat_v[j // 8, pl.ds((j % 8) * 16, 16)] = d16 * B + s16
    for r in range(4):
        pltpu.sync_copy(ones_v, c_sh.at[flat_v.at[r]], add=True)
    plsc.subcore_barrier()
    pltpu.sync_copy(c_sh.at[pl.ds(s * seg, seg)],
                    out_hbm.at[c, pl.ds(s * seg, seg)])


def _build_counts(psrc, pdst):
    mesh = plsc.VectorSubcoreMesh(core_axis_name="c", subcore_axis_name="s")
    k = functools.partial(
        pl.kernel, mesh=mesh,
        out_type=jax.ShapeDtypeStruct((2, B * B), jnp.float32),
        scratch_types=[
            pltpu.VMEM((E // 32,), jnp.int32),
            pltpu.VMEM((E // 32,), jnp.int32),
            pltpu.VMEM((4, 128), jnp.int32),
            pltpu.VMEM((128,), jnp.float32),
            pltpu.VMEM_SHARED((B * B,), jnp.float32),
        ],
    )(_count_body)
    return k(psrc, pdst, jnp.zeros((262144 // 16,), jnp.float32))


def _patient_body(cpair_ref, mrow_ref, mcol_ref, hN_ref,
                  Wc_ref, bc_ref, Wl1_ref, bl1_ref, Wl2_ref, bl2_ref,
                  out_ref):
    C = cpair_ref[0] + cpair_ref[1]

    degS = jnp.sum(C, axis=0, keepdims=True)
    degD = jnp.sum(C, axis=1, keepdims=True)
    ns = jax.lax.rsqrt(jnp.maximum(degS, 1.0))
    nd = jax.lax.rsqrt(jnp.maximum(degD, 1.0))
    A = C * mcol_ref[...] * (mrow_ref[...] * ns)

    hN = hN_ref[...]
    hcur = hN
    hsum = hN
    for i in range(3):
        agg = jnp.dot(A, hcur)
        rst = jnp.dot(agg * nd, Wc_ref[i]) + bc_ref[i, :][None, :]
        hcur = jnp.maximum(BN_SCALE * rst, 0.0) + hcur
        hsum = hsum + hcur
    hm = hsum * 0.25

    z = jnp.dot(hm, Wl1_ref[...]) + bl1_ref[...][None, :]
    mu = jnp.mean(z, axis=-1, keepdims=True)
    var = jnp.mean((z - mu) ** 2, axis=-1, keepdims=True)
    zn = jnp.maximum((z - mu) * jax.lax.rsqrt(var + LN_EPS), 0.0)
    out_ref[...] = jnp.dot(zn, Wl2_ref[...]) + bl2_ref[...][None, :]


def kernel(plane_feats, plane_edge_index, patient_edge_index,
           original_features, mask, W1, a1s, a1d, res1, b1, W2, a2s, a2d,
           res2, b2, D1, d1b, D2, d2b, Wf, bf, Wc, bc, Wl1, bl1, Wl2, bl2):
    f32 = jnp.float32
    xk = plane_feats.reshape(B, K, P).transpose(1, 0, 2)
    xflat = xk.reshape(K, B * P)
    ef = plane_edge_index.astype(f32)
    idx8 = jnp.zeros((8, 2 * EP), f32)
    idx8 = idx8.at[0:2].set(jnp.tile(ef, (1, 2)))
    idx8 = idx8.at[2, EP:].set(1.0)

    W1k = W1[:, 0, :]
    res1k = res1[:, 0, :]
    w1s = jnp.sum(W1k.reshape(K, HEADS, HID) * a1s, axis=-1)
    w1d = jnp.sum(W1k.reshape(K, HEADS, HID) * a1d, axis=-1)
    wA = jnp.repeat(w1s, EP, axis=1)
    wB = jnp.repeat(w1d, EP, axis=1)
    hsel = (jnp.arange(HEADS * HID) >= HID).astype(f32)
    Wg = jnp.stack([W1k * (1.0 - hsel), W1k * hsel, res1k], axis=1)
    w2s = jnp.einsum('kfo,ko->kf', W2, a2s[:, 0])
    w2d = jnp.einsum('kfo,ko->kf', W2, a2d[:, 0])
    W2w = jnp.concatenate([W2, res2], axis=2)
    D2f = D2[:, :, 0]
    d2b8 = jnp.broadcast_to(d2b, (K, 8)) if d2b.shape[1] == 1 else d2b
    Wfo = Wf[:ORIG]
    Wfg = Wf[ORIG:].reshape(K, OUT, NH)

    nblk = B // BB
    grid_spec = pl.GridSpec(
        grid=(nblk,),
        in_specs=[
            pl.BlockSpec((8, 2 * EP), lambda i: (0, 0)),
            pl.BlockSpec((K, BB, P), lambda i: (0, i, 0)),
            pl.BlockSpec((K, BB * P), lambda i: (0, i)),
            pl.BlockSpec((BB, ORIG), lambda i: (i, 0)),
            pl.BlockSpec((K, 3, NH), lambda i: (0, 0, 0)),
            pl.BlockSpec((K, NH), lambda i: (0, 0)),
            pl.BlockSpec((K, 2 * EP), lambda i: (0, 0)),
            pl.BlockSpec((K, 2 * EP), lambda i: (0, 0)),
            pl.BlockSpec((K, NH), lambda i: (0, 0)),
            pl.BlockSpec((K, NH), lambda i: (0, 0)),
            pl.BlockSpec((K, NH, 2 * OUT), lambda i: (0, 0, 0)),
            pl.BlockSpec((K, OUT), lambda i: (0, 0)),
            pl.BlockSpec((K, OUT, NH), lambda i: (0, 0, 0)),
            pl.BlockSpec((K, NH), lambda i: (0, 0)),
            pl.BlockSpec((K, NH), lambda i: (0, 0)),
            pl.BlockSpec((K, 8), lambda i: (0, 0)),
            pl.BlockSpec((ORIG, NH), lambda i: (0, 0)),
            pl.BlockSpec((K, OUT, NH), lambda i: (0, 0, 0)),
            pl.BlockSpec((NH,), lambda i: (0,)),
        ],
        out_specs=[
            pl.BlockSpec((BB, NH), lambda i: (i, 0)),
            pl.BlockSpec((1, BB * P), lambda i: (0, 0)),
        ],
    )
    hN, loss = _pc(
        _plane_body,
        grid_spec=grid_spec,
        out_shape=[jax.ShapeDtypeStruct((B, NH), f32),
                   jax.ShapeDtypeStruct((1, BB * P), f32)],
    )(idx8, xk, xflat, original_features, Wg, b1, wA, wB, w2s, w2d, W2w, b2,
      D1, d1b, D2f, d2b8, Wfo, Wfg, bf)

    cpair = _build_counts(patient_edge_index[0], patient_edge_index[1])
    cpair = cpair.reshape(2, B, B)
    maskf = mask.astype(f32)
    mrow = maskf.reshape(1, B)
    mcol = maskf.reshape(B, 1)

    logits = _pc(
        _patient_body,
        out_shape=jax.ShapeDtypeStruct((B, NCLS), f32),
    )(cpair, mrow, mcol, hN, Wc, bc, Wl1, bl1, Wl2, bl2)

    return logits, jnp.sum(loss)


# speedup vs baseline: 1.0288x; 1.0063x over previous
import functools
import jax
import jax.numpy as jnp
import numpy as np
from jax import lax
from jax.experimental import pallas as pl
from jax.experimental.pallas import tpu as pltpu
from jax.experimental.pallas import tpu_sc as plsc

B = 512; K = 24; P = 16; EP = 32; E = 16384
FIN = 1; HID = 64; HEADS = 2; OUT = 32; NH = 128; ORIG = 256; NCLS = 2
BN_SCALE = 1.0 / np.sqrt(1.0 + 1e-5)
LN_EPS = 1e-5
BB = 64

_pc = pl.pallas_call


def _leaky(x):
    return jnp.where(x >= 0, x, 0.2 * x)


def _plane_body(idx_ref, x_ref, xf_ref, orig_ref, Wg_ref,
                wA_ref, wB_ref, w2s_ref, w2d_ref, W2_ref, b2_ref,
                D1_ref, D2f_ref, d2b_ref, Wfo_ref, Wfg_ref, bf_ref,
                hN_ref, loss_ref):
    src = idx_ref[0:1, 0:EP]
    dst = idx_ref[1:2, 0:EP]
    pio = jax.lax.broadcasted_iota(jnp.int32, (P, EP), 0).astype(jnp.float32)
    Gs = (pio == src).astype(jnp.float32)
    Gd = (pio == dst).astype(jnp.float32)
    src2 = idx_ref[0:1, :]
    dst2 = idx_ref[1:2, :]
    hrow = idx_ref[2:3, :]
    rio = jax.lax.broadcasted_iota(jnp.int32, (2 * P, 2 * EP), 0)
    rp = jnp.remainder(rio, P).astype(jnp.float32)
    rh = (rio // P).astype(jnp.float32)
    Gs2 = (rp == src2).astype(jnp.float32) * (rh == hrow)
    Gd2 = (rp == dst2).astype(jnp.float32) * (rh == hrow)

    x = x_ref[...]
    b2 = b2_ref[...]
    d2b = d2b_ref[...]
    xs2 = jax.lax.dot_general(x, Gs2[0:P] + Gs2[P:2 * P],
                              (((2,), (0,)), ((), ())))
    xd2 = jax.lax.dot_general(x, Gd2[0:P] + Gd2[P:2 * P],
                              (((2,), (0,)), ((), ())))
    wA = wA_ref[...]
    wB = wB_ref[...]
    e12 = _leaky(xs2 * wA[:, None, :] + xd2 * wB[:, None, :])
    m = jnp.max(e12, axis=2, keepdims=True)
    a12 = jnp.exp(e12 - m)
    den12 = jax.lax.dot_general(a12, Gd2, (((2,), (1,)), ((), ())))
    num12 = jax.lax.dot_general(a12 * xs2, Gd2, (((2,), (1,)), ((), ())))
    s1p = jnp.where(den12 > 0, num12 / jnp.maximum(den12, 1e-30), 0.0)
    s1_0 = s1p[:, :, 0:P]
    s1_1 = s1p[:, :, P:2 * P]
    G = jnp.concatenate([s1_0[..., None], s1_1[..., None], x[..., None],
                         jnp.ones((K, BB, P, 1), jnp.float32)],
                        axis=-1).reshape(K, BB * P, 4)
    h1r = jnp.maximum(
        BN_SCALE * jax.lax.dot_general(G, Wg_ref[...],
                                       (((2,), (1,)), ((0,), (0,)))),
        0.0)
    h1 = h1r.reshape(K, BB, P, HEADS * HID)

    feat2w = jax.lax.dot_general(h1r, W2_ref[...],
                                 (((2,), (1,)), ((0,), (0,))))
    feat2 = feat2w[:, :, 0:OUT]
    el2 = jnp.sum(h1 * w2s_ref[...][:, None, None, :], axis=-1)
    er2 = jnp.sum(h1 * w2d_ref[...][:, None, None, :], axis=-1)
    e2 = _leaky(jax.lax.dot_general(el2, Gs, (((2,), (0,)), ((), ())))
                + jax.lax.dot_general(er2, Gd, (((2,), (0,)), ((), ()))))
    m2 = jnp.max(e2, axis=2, keepdims=True)
    a2 = jnp.exp(e2 - m2)
    den2 = jax.lax.dot_general(a2, Gd, (((2,), (1,)), ((), ())))
    A1 = (Gd[None, None] * a2[:, :, None, :]).reshape(K * BB, P, EP)
    M2 = jax.lax.dot_general(A1, Gs, (((2,), (1,)), ((), ())))
    feat2b = feat2.reshape(K * BB, P, OUT)
    num2 = jax.lax.dot_general(M2, feat2b,
                               (((2,), (1,)), ((0,), (0,))))
    inv2 = jnp.where(den2 > 0, 1.0 / jnp.maximum(den2, 1e-30), 0.0)
    out2 = num2 * inv2.reshape(K * BB, P, 1)
    res2o = feat2w[:, :, OUT:2 * OUT]
    h2 = jnp.maximum(
        BN_SCALE * ((out2.reshape(K, BB, P, OUT)
                     + res2o.reshape(K, BB, P, OUT))
                    + b2[:, None, None, :]), 0.0)

    graph_rep = jnp.mean(h2, axis=2)

    h2r = jnp.concatenate([h2, jnp.ones((K, BB, P, 1), jnp.float32)],
                          axis=-1).reshape(K, BB * P, OUT + 1)
    d1 = jnp.maximum(
        BN_SCALE * jax.lax.dot_general(h2r, D1_ref[...],
                                       (((2,), (1,)), ((0,), (0,)))),
        0.0)
    rec = jax.lax.dot_general(d1, D2f_ref[...],
                              (((2,), (1,)), ((0,), (0,)))) + d2b[:, 0:1]
    xr = xf_ref[...]
    sq = (rec - xr) ** 2
    partial = jnp.sum(sq, axis=0, keepdims=True) * (1.0 / (P * B * K))

    @pl.when(pl.program_id(0) == 0)
    def _():
        loss_ref[...] = jnp.zeros((1, BB * P), jnp.float32)
    loss_ref[...] += partial

    g = jax.lax.dot_general(graph_rep, Wfg_ref[...],
                            (((2,), (1,)), ((0,), (0,))))
    gsum = jnp.sum(g, axis=0)
    ho = jnp.dot(orig_ref[...], Wfo_ref[...])
    hN_ref[...] = jnp.maximum(BN_SCALE * (gsum + ho + bf_ref[...][None, :]),
                              0.0)


def _count_body(ps_hbm, pd_hbm, z_hbm, out_hbm, ps_v, pd_v, flat_v, ones_v,
                c_sh):
    c = lax.axis_index("c")
    s = lax.axis_index("s")
    wid = s * 2 + c
    seg = 262144 // 16
    pltpu.sync_copy(z_hbm, c_sh.at[pl.ds(s * seg, seg)])
    plsc.subcore_barrier()
    base = wid * (E // 32)
    pltpu.sync_copy(ps_hbm.at[pl.ds(base, E // 32)], ps_v)
    pltpu.sync_copy(pd_hbm.at[pl.ds(base, E // 32)], pd_v)
    for j in range(8):
        ones_v[pl.ds(j * 16, 16)] = jnp.ones((16,), jnp.float32)
    for j in range(32):
        s16 = ps_v[pl.ds(j * 16, 16)]
        d16 = pd_v[pl.ds(j * 16, 16)]
        flat_v[j // 8, pl.ds((j % 8) * 16, 16)] = d16 * B + s16
    for r in range(4):
        pltpu.sync_copy(ones_v, c_sh.at[flat_v.at[r]], add=True)
    plsc.subcore_barrier()
    pltpu.sync_copy(c_sh.at[pl.ds(s * seg, seg)],
                    out_hbm.at[c, pl.ds(s * seg, seg)])


def _build_counts(psrc, pdst):
    mesh = plsc.VectorSubcoreMesh(core_axis_name="c", subcore_axis_name="s")
    k = functools.partial(
        pl.kernel, mesh=mesh,
        out_type=jax.ShapeDtypeStruct((2, B * B), jnp.float32),
        scratch_types=[
            pltpu.VMEM((E // 32,), jnp.int32),
            pltpu.VMEM((E // 32,), jnp.int32),
            pltpu.VMEM((4, 128), jnp.int32),
            pltpu.VMEM((128,), jnp.float32),
            pltpu.VMEM_SHARED((B * B,), jnp.float32),
        ],
    )(_count_body)
    return k(psrc, pdst, jnp.zeros((262144 // 16,), jnp.float32))


def _patient_body(cpair_ref, mrow_ref, mcol_ref, hN_ref,
                  Wc_ref, bc_ref, Wl1_ref, bl1_ref, Wl2_ref, bl2_ref,
                  out_ref):
    C = cpair_ref[0] + cpair_ref[1]

    degS = jnp.sum(C, axis=0, keepdims=True)
    degD = jnp.sum(C, axis=1, keepdims=True)
    ns = jax.lax.rsqrt(jnp.maximum(degS, 1.0))
    nd = jax.lax.rsqrt(jnp.maximum(degD, 1.0))
    A = C * mcol_ref[...] * (mrow_ref[...] * ns)

    hN = hN_ref[...]
    hcur = hN
    hsum = hN
    for i in range(3):
        agg = jnp.dot(A, hcur)
        rst = jnp.dot(agg * nd, Wc_ref[i]) + bc_ref[i, :][None, :]
        hcur = jnp.maximum(BN_SCALE * rst, 0.0) + hcur
        hsum = hsum + hcur
    hm = hsum * 0.25

    z = jnp.dot(hm, Wl1_ref[...]) + bl1_ref[...][None, :]
    mu = jnp.mean(z, axis=-1, keepdims=True)
    var = jnp.mean((z - mu) ** 2, axis=-1, keepdims=True)
    zn = jnp.maximum((z - mu) * jax.lax.rsqrt(var + LN_EPS), 0.0)
    out_ref[...] = jnp.dot(zn, Wl2_ref[...]) + bl2_ref[...][None, :]


def kernel(plane_feats, plane_edge_index, patient_edge_index,
           original_features, mask, W1, a1s, a1d, res1, b1, W2, a2s, a2d,
           res2, b2, D1, d1b, D2, d2b, Wf, bf, Wc, bc, Wl1, bl1, Wl2, bl2):
    f32 = jnp.float32
    xk = plane_feats.reshape(B, K, P).transpose(1, 0, 2)
    xflat = xk.reshape(K, B * P)
    ef = plane_edge_index.astype(f32)
    idx8 = jnp.zeros((8, 2 * EP), f32)
    idx8 = idx8.at[0:2].set(jnp.tile(ef, (1, 2)))
    idx8 = idx8.at[2, EP:].set(1.0)

    W1k = W1[:, 0, :]
    res1k = res1[:, 0, :]
    w1s = jnp.sum(W1k.reshape(K, HEADS, HID) * a1s, axis=-1)
    w1d = jnp.sum(W1k.reshape(K, HEADS, HID) * a1d, axis=-1)
    wA = jnp.repeat(w1s, EP, axis=1)
    wB = jnp.repeat(w1d, EP, axis=1)
    hsel = (jnp.arange(HEADS * HID) >= HID).astype(f32)
    Wg = jnp.stack([W1k * (1.0 - hsel), W1k * hsel, res1k, b1], axis=1)
    D1e = jnp.concatenate([D1, d1b[:, None, :]], axis=1)
    w2s = jnp.einsum('kfo,ko->kf', W2, a2s[:, 0])
    w2d = jnp.einsum('kfo,ko->kf', W2, a2d[:, 0])
    W2w = jnp.concatenate([W2, res2], axis=2)
    D2f = D2[:, :, 0]
    d2b8 = jnp.broadcast_to(d2b, (K, 8)) if d2b.shape[1] == 1 else d2b
    Wfo = Wf[:ORIG]
    Wfg = Wf[ORIG:].reshape(K, OUT, NH)

    nblk = B // BB
    grid_spec = pl.GridSpec(
        grid=(nblk,),
        in_specs=[
            pl.BlockSpec((8, 2 * EP), lambda i: (0, 0)),
            pl.BlockSpec((K, BB, P), lambda i: (0, i, 0)),
            pl.BlockSpec((K, BB * P), lambda i: (0, i)),
            pl.BlockSpec((BB, ORIG), lambda i: (i, 0)),
            pl.BlockSpec((K, 4, NH), lambda i: (0, 0, 0)),
            pl.BlockSpec((K, 2 * EP), lambda i: (0, 0)),
            pl.BlockSpec((K, 2 * EP), lambda i: (0, 0)),
            pl.BlockSpec((K, NH), lambda i: (0, 0)),
            pl.BlockSpec((K, NH), lambda i: (0, 0)),
            pl.BlockSpec((K, NH, 2 * OUT), lambda i: (0, 0, 0)),
            pl.BlockSpec((K, OUT), lambda i: (0, 0)),
            pl.BlockSpec((K, OUT + 1, NH), lambda i: (0, 0, 0)),
            pl.BlockSpec((K, NH), lambda i: (0, 0)),
            pl.BlockSpec((K, 8), lambda i: (0, 0)),
            pl.BlockSpec((ORIG, NH), lambda i: (0, 0)),
            pl.BlockSpec((K, OUT, NH), lambda i: (0, 0, 0)),
            pl.BlockSpec((NH,), lambda i: (0,)),
        ],
        out_specs=[
            pl.BlockSpec((BB, NH), lambda i: (i, 0)),
            pl.BlockSpec((1, BB * P), lambda i: (0, 0)),
        ],
    )
    hN, loss = _pc(
        _plane_body,
        grid_spec=grid_spec,
        out_shape=[jax.ShapeDtypeStruct((B, NH), f32),
                   jax.ShapeDtypeStruct((1, BB * P), f32)],
    )(idx8, xk, xflat, original_features, Wg, wA, wB, w2s, w2d, W2w, b2,
      D1e, D2f, d2b8, Wfo, Wfg, bf)

    cpair = _build_counts(patient_edge_index[0], patient_edge_index[1])
    cpair = cpair.reshape(2, B, B)
    maskf = mask.astype(f32)
    mrow = maskf.reshape(1, B)
    mcol = maskf.reshape(B, 1)

    logits = _pc(
        _patient_body,
        out_shape=jax.ShapeDtypeStruct((B, NCLS), f32),
    )(cpair, mrow, mcol, hN, Wc, bc, Wl1, bl1, Wl2, bl2)

    return logits, jnp.sum(loss)


# speedup vs baseline: 1.2089x; 1.1750x over previous
import functools
import jax
import jax.numpy as jnp
import numpy as np
from jax import lax
from jax.experimental import pallas as pl
from jax.experimental.pallas import tpu as pltpu
from jax.experimental.pallas import tpu_sc as plsc

B = 512; K = 24; P = 16; EP = 32; E = 16384
FIN = 1; HID = 64; HEADS = 2; OUT = 32; NH = 128; ORIG = 256; NCLS = 2
BN_SCALE = 1.0 / np.sqrt(1.0 + 1e-5)
LN_EPS = 1e-5
BB = 64

_pc = pl.pallas_call


def _leaky(x):
    return jnp.where(x >= 0, x, 0.2 * x)


def _plane_body(idx_ref, x_ref, xf_ref, orig_ref, Wg_ref,
                wA_ref, wB_ref, W2_ref, b2_ref,
                D1_ref, D2f_ref, d2b_ref, Wfo_ref, Wfg_ref, bf_ref,
                hN_ref, loss_ref):
    src = idx_ref[0:1, 0:EP]
    dst = idx_ref[1:2, 0:EP]
    pio = jax.lax.broadcasted_iota(jnp.int32, (P, EP), 0).astype(jnp.float32)
    Gs = (pio == src).astype(jnp.float32)
    Gd = (pio == dst).astype(jnp.float32)
    src2 = idx_ref[0:1, :]
    dst2 = idx_ref[1:2, :]
    hrow = idx_ref[2:3, :]
    rio = jax.lax.broadcasted_iota(jnp.int32, (2 * P, 2 * EP), 0)
    rp = jnp.remainder(rio, P).astype(jnp.float32)
    rh = (rio // P).astype(jnp.float32)
    Gs2 = (rp == src2).astype(jnp.float32) * (rh == hrow)
    Gd2 = (rp == dst2).astype(jnp.float32) * (rh == hrow)

    x = x_ref[...]
    b2 = b2_ref[...]
    d2b = d2b_ref[...]
    xs2 = jax.lax.dot_general(x, Gs2[0:P] + Gs2[P:2 * P],
                              (((2,), (0,)), ((), ())))
    xd2 = jax.lax.dot_general(x, Gd2[0:P] + Gd2[P:2 * P],
                              (((2,), (0,)), ((), ())))
    wA = wA_ref[...]
    wB = wB_ref[...]
    e12 = _leaky(xs2 * wA[:, None, :] + xd2 * wB[:, None, :])
    m = jnp.max(e12, axis=2, keepdims=True)
    a12 = jnp.exp(e12 - m)
    den12 = jax.lax.dot_general(a12, Gd2, (((2,), (1,)), ((), ())))
    num12 = jax.lax.dot_general(a12 * xs2, Gd2, (((2,), (1,)), ((), ())))
    s1p = jnp.where(den12 > 0, num12 / jnp.maximum(den12, 1e-30), 0.0)
    s1_0 = s1p[:, :, 0:P]
    s1_1 = s1p[:, :, P:2 * P]
    G = jnp.concatenate([s1_0[..., None], s1_1[..., None], x[..., None],
                         jnp.ones((K, BB, P, 1), jnp.float32)],
                        axis=-1).reshape(K, BB * P, 4)
    h1r = jnp.maximum(
        BN_SCALE * jax.lax.dot_general(G, Wg_ref[...],
                                       (((2,), (1,)), ((0,), (0,)))),
        0.0)
    h1 = h1r.reshape(K, BB, P, HEADS * HID)

    feat2w = jax.lax.dot_general(h1r, W2_ref[...],
                                 (((2,), (1,)), ((0,), (0,))))
    feat2 = feat2w[:, :, 0:OUT]
    el2 = feat2w[:, :, 2 * OUT].reshape(K, BB, P)
    er2 = feat2w[:, :, 2 * OUT + 1].reshape(K, BB, P)
    e2 = _leaky(jax.lax.dot_general(el2, Gs, (((2,), (0,)), ((), ())))
                + jax.lax.dot_general(er2, Gd, (((2,), (0,)), ((), ()))))
    m2 = jnp.max(e2, axis=2, keepdims=True)
    a2 = jnp.exp(e2 - m2)
    den2 = jax.lax.dot_general(a2, Gd, (((2,), (1,)), ((), ())))
    A1 = (Gd[None, None] * a2[:, :, None, :]).reshape(K * BB, P, EP)
    M2 = jax.lax.dot_general(A1, Gs, (((2,), (1,)), ((), ())))
    feat2b = feat2.reshape(K * BB, P, OUT)
    num2 = jax.lax.dot_general(M2, feat2b,
                               (((2,), (1,)), ((0,), (0,))))
    inv2 = jnp.where(den2 > 0, 1.0 / jnp.maximum(den2, 1e-30), 0.0)
    out2 = num2 * inv2.reshape(K * BB, P, 1)
    res2o = feat2w[:, :, OUT:2 * OUT]
    h2 = jnp.maximum(
        BN_SCALE * ((out2.reshape(K, BB, P, OUT)
                     + res2o.reshape(K, BB, P, OUT))
                    + b2[:, None, None, :]), 0.0)

    graph_rep = jnp.mean(h2, axis=2)

    h2r = jnp.concatenate([h2, jnp.ones((K, BB, P, 1), jnp.float32)],
                          axis=-1).reshape(K, BB * P, OUT + 1)
    d1 = jnp.maximum(
        BN_SCALE * jax.lax.dot_general(h2r, D1_ref[...],
                                       (((2,), (1,)), ((0,), (0,)))),
        0.0)
    rec = jax.lax.dot_general(d1, D2f_ref[...],
                              (((2,), (1,)), ((0,), (0,)))) + d2b[:, 0:1]
    xr = xf_ref[...]
    sq = (rec - xr) ** 2
    partial = jnp.sum(sq, axis=0, keepdims=True) * (1.0 / (P * B * K))

    @pl.when(pl.program_id(0) == 0)
    def _():
        loss_ref[...] = jnp.zeros((1, BB * P), jnp.float32)
    loss_ref[...] += partial

    g = jax.lax.dot_general(graph_rep, Wfg_ref[...],
                            (((2,), (1,)), ((0,), (0,))))
    gsum = jnp.sum(g, axis=0)
    ho = jnp.dot(orig_ref[...], Wfo_ref[...])
    hN_ref[...] = jnp.maximum(BN_SCALE * (gsum + ho + bf_ref[...][None, :]),
                              0.0)


def _count_body(ps_hbm, pd_hbm, z_hbm, out_hbm, ps_v, pd_v, flat_v, ones_v,
                c_sh):
    c = lax.axis_index("c")
    s = lax.axis_index("s")
    wid = s * 2 + c
    seg = 262144 // 16
    pltpu.sync_copy(z_hbm, c_sh.at[pl.ds(s * seg, seg)])
    plsc.subcore_barrier()
    base = wid * (E // 32)
    pltpu.sync_copy(ps_hbm.at[pl.ds(base, E // 32)], ps_v)
    pltpu.sync_copy(pd_hbm.at[pl.ds(base, E // 32)], pd_v)
    for j in range(8):
        ones_v[pl.ds(j * 16, 16)] = jnp.ones((16,), jnp.float32)
    for j in range(32):
        s16 = ps_v[pl.ds(j * 16, 16)]
        d16 = pd_v[pl.ds(j * 16, 16)]
        flat_v[j // 8, pl.ds((j % 8) * 16, 16)] = d16 * B + s16
    for r in range(4):
        pltpu.sync_copy(ones_v, c_sh.at[flat_v.at[r]], add=True)
    plsc.subcore_barrier()
    pltpu.sync_copy(c_sh.at[pl.ds(s * seg, seg)],
                    out_hbm.at[c, pl.ds(s * seg, seg)])


def _build_counts(psrc, pdst):
    mesh = plsc.VectorSubcoreMesh(core_axis_name="c", subcore_axis_name="s")
    k = functools.partial(
        pl.kernel, mesh=mesh,
        out_type=jax.ShapeDtypeStruct((2, B * B), jnp.float32),
        scratch_types=[
            pltpu.VMEM((E // 32,), jnp.int32),
            pltpu.VMEM((E // 32,), jnp.int32),
            pltpu.VMEM((4, 128), jnp.int32),
            pltpu.VMEM((128,), jnp.float32),
            pltpu.VMEM_SHARED((B * B,), jnp.float32),
        ],
    )(_count_body)
    return k(psrc, pdst, jnp.zeros((262144 // 16,), jnp.float32))


def _patient_body(cpair_ref, mrow_ref, mcol_ref, hN_ref,
                  Wc_ref, bc_ref, Wl1_ref, bl1_ref, Wl2_ref, bl2_ref,
                  out_ref):
    C = cpair_ref[0] + cpair_ref[1]

    degS = jnp.sum(C, axis=0, keepdims=True)
    degD = jnp.sum(C, axis=1, keepdims=True)
    ns = jax.lax.rsqrt(jnp.maximum(degS, 1.0))
    nd = jax.lax.rsqrt(jnp.maximum(degD, 1.0))
    A = C * mcol_ref[...] * (mrow_ref[...] * ns)

    hN = hN_ref[...]
    hcur = hN
    hsum = hN
    for i in range(3):
        agg = jnp.dot(A, hcur)
        rst = jnp.dot(agg * nd, Wc_ref[i]) + bc_ref[i, :][None, :]
        hcur = jnp.maximum(BN_SCALE * rst, 0.0) + hcur
        hsum = hsum + hcur
    hm = hsum * 0.25

    z = jnp.dot(hm, Wl1_ref[...]) + bl1_ref[...][None, :]
    mu = jnp.mean(z, axis=-1, keepdims=True)
    var = jnp.mean((z - mu) ** 2, axis=-1, keepdims=True)
    zn = jnp.maximum((z - mu) * jax.lax.rsqrt(var + LN_EPS), 0.0)
    out_ref[...] = jnp.dot(zn, Wl2_ref[...]) + bl2_ref[...][None, :]


def kernel(plane_feats, plane_edge_index, patient_edge_index,
           original_features, mask, W1, a1s, a1d, res1, b1, W2, a2s, a2d,
           res2, b2, D1, d1b, D2, d2b, Wf, bf, Wc, bc, Wl1, bl1, Wl2, bl2):
    f32 = jnp.float32
    xk = plane_feats.reshape(B, K, P).transpose(1, 0, 2)
    xflat = xk.reshape(K, B * P)
    ef = plane_edge_index.astype(f32)
    idx8 = jnp.zeros((8, 2 * EP), f32)
    idx8 = idx8.at[0:2].set(jnp.tile(ef, (1, 2)))
    idx8 = idx8.at[2, EP:].set(1.0)

    W1k = W1[:, 0, :]
    res1k = res1[:, 0, :]
    w1s = jnp.sum(W1k.reshape(K, HEADS, HID) * a1s, axis=-1)
    w1d = jnp.sum(W1k.reshape(K, HEADS, HID) * a1d, axis=-1)
    wA = jnp.repeat(w1s, EP, axis=1)
    wB = jnp.repeat(w1d, EP, axis=1)
    hsel = (jnp.arange(HEADS * HID) >= HID).astype(f32)
    Wg = jnp.stack([W1k * (1.0 - hsel), W1k * hsel, res1k, b1], axis=1)
    D1e = jnp.concatenate([D1, d1b[:, None, :]], axis=1)
    w2s = jnp.einsum('kfo,ko->kf', W2, a2s[:, 0])
    w2d = jnp.einsum('kfo,ko->kf', W2, a2d[:, 0])
    W2w = jnp.concatenate([W2, res2, w2s[:, :, None], w2d[:, :, None],
                           jnp.zeros((K, NH, 6), f32)], axis=2)
    D2f = D2[:, :, 0]
    d2b8 = jnp.broadcast_to(d2b, (K, 8)) if d2b.shape[1] == 1 else d2b
    Wfo = Wf[:ORIG]
    Wfg = Wf[ORIG:].reshape(K, OUT, NH)

    nblk = B // BB
    grid_spec = pl.GridSpec(
        grid=(nblk,),
        in_specs=[
            pl.BlockSpec((8, 2 * EP), lambda i: (0, 0)),
            pl.BlockSpec((K, BB, P), lambda i: (0, i, 0)),
            pl.BlockSpec((K, BB * P), lambda i: (0, i)),
            pl.BlockSpec((BB, ORIG), lambda i: (i, 0)),
            pl.BlockSpec((K, 4, NH), lambda i: (0, 0, 0)),
            pl.BlockSpec((K, 2 * EP), lambda i: (0, 0)),
            pl.BlockSpec((K, 2 * EP), lambda i: (0, 0)),
            pl.BlockSpec((K, NH, 2 * OUT + 8), lambda i: (0, 0, 0)),
            pl.BlockSpec((K, OUT), lambda i: (0, 0)),
            pl.BlockSpec((K, OUT + 1, NH), lambda i: (0, 0, 0)),
            pl.BlockSpec((K, NH), lambda i: (0, 0)),
            pl.BlockSpec((K, 8), lambda i: (0, 0)),
            pl.BlockSpec((ORIG, NH), lambda i: (0, 0)),
            pl.BlockSpec((K, OUT, NH), lambda i: (0, 0, 0)),
            pl.BlockSpec((NH,), lambda i: (0,)),
        ],
        out_specs=[
            pl.BlockSpec((BB, NH), lambda i: (i, 0)),
            pl.BlockSpec((1, BB * P), lambda i: (0, 0)),
        ],
    )
    hN, loss = _pc(
        _plane_body,
        grid_spec=grid_spec,
        out_shape=[jax.ShapeDtypeStruct((B, NH), f32),
                   jax.ShapeDtypeStruct((1, BB * P), f32)],
    )(idx8, xk, xflat, original_features, Wg, wA, wB, W2w, b2,
      D1e, D2f, d2b8, Wfo, Wfg, bf)

    cpair = _build_counts(patient_edge_index[0], patient_edge_index[1])
    cpair = cpair.reshape(2, B, B)
    maskf = mask.astype(f32)
    mrow = maskf.reshape(1, B)
    mcol = maskf.reshape(B, 1)

    logits = _pc(
        _patient_body,
        out_shape=jax.ShapeDtypeStruct((B, NCLS), f32),
    )(cpair, mrow, mcol, hN, Wc, bc, Wl1, bl1, Wl2, bl2)

    return logits, jnp.sum(loss)


# speedup vs baseline: 1.2090x; 1.0002x over previous
import functools
import jax
import jax.numpy as jnp
import numpy as np
from jax import lax
from jax.experimental import pallas as pl
from jax.experimental.pallas import tpu as pltpu
from jax.experimental.pallas import tpu_sc as plsc

B = 512; K = 24; P = 16; EP = 32; E = 16384
FIN = 1; HID = 64; HEADS = 2; OUT = 32; NH = 128; ORIG = 256; NCLS = 2
BN_SCALE = 1.0 / np.sqrt(1.0 + 1e-5)
LN_EPS = 1e-5
BB = 64

_pc = pl.pallas_call


def _leaky(x):
    return jnp.where(x >= 0, x, 0.2 * x)


def _plane_body(idx_ref, x_ref, xf_ref, orig_ref, Wg_ref,
                wA_ref, wB_ref, W2_ref, b2_ref,
                D1_ref, D2f_ref, d2b_ref, Wfo_ref, Wfg_ref, bf_ref,
                hN_ref, loss_ref):
    src = idx_ref[0:1, 0:EP]
    dst = idx_ref[1:2, 0:EP]
    pio = jax.lax.broadcasted_iota(jnp.int32, (P, EP), 0).astype(jnp.float32)
    Gs = (pio == src).astype(jnp.float32)
    Gd = (pio == dst).astype(jnp.float32)
    src2 = idx_ref[0:1, :]
    dst2 = idx_ref[1:2, :]
    hrow = idx_ref[2:3, :]
    rio = jax.lax.broadcasted_iota(jnp.int32, (2 * P, 2 * EP), 0)
    rp = jnp.remainder(rio, P).astype(jnp.float32)
    rh = (rio // P).astype(jnp.float32)
    Gs2 = (rp == src2).astype(jnp.float32) * (rh == hrow)
    Gd2 = (rp == dst2).astype(jnp.float32) * (rh == hrow)

    x = x_ref[...]
    b2 = b2_ref[...]
    d2b = d2b_ref[...]
    xs2 = jax.lax.dot_general(x, Gs2[0:P] + Gs2[P:2 * P],
                              (((2,), (0,)), ((), ())))
    xd2 = jax.lax.dot_general(x, Gd2[0:P] + Gd2[P:2 * P],
                              (((2,), (0,)), ((), ())))
    wA = wA_ref[...]
    wB = wB_ref[...]
    e12 = _leaky(xs2 * wA[:, None, :] + xd2 * wB[:, None, :])
    m = jnp.max(e12, axis=2, keepdims=True)
    a12 = jnp.exp(e12 - m)
    den12 = jax.lax.dot_general(a12, Gd2, (((2,), (1,)), ((), ())))
    num12 = jax.lax.dot_general(a12 * xs2, Gd2, (((2,), (1,)), ((), ())))
    s1p = jnp.where(den12 > 0, num12 / jnp.maximum(den12, 1e-30), 0.0)
    s1_0 = s1p[:, :, 0:P]
    s1_1 = s1p[:, :, P:2 * P]
    G = jnp.concatenate([s1_0[..., None], s1_1[..., None], x[..., None],
                         jnp.ones((K, BB, P, 1), jnp.float32)],
                        axis=-1).reshape(K, BB * P, 4)
    h1r = jnp.maximum(
        BN_SCALE * jax.lax.dot_general(G, Wg_ref[...],
                                       (((2,), (1,)), ((0,), (0,)))),
        0.0)

    feat2w = jax.lax.dot_general(h1r, W2_ref[...],
                                 (((2,), (1,)), ((0,), (0,))))
    feat2 = feat2w[:, :, 0:OUT]
    el2 = feat2w[:, :, 2 * OUT].reshape(K, BB, P)
    er2 = feat2w[:, :, 2 * OUT + 1].reshape(K, BB, P)
    e2 = _leaky(jax.lax.dot_general(el2, Gs, (((2,), (0,)), ((), ())))
                + jax.lax.dot_general(er2, Gd, (((2,), (0,)), ((), ()))))
    m2 = jnp.max(e2, axis=2, keepdims=True)
    a2 = jnp.exp(e2 - m2)
    den2 = jax.lax.dot_general(a2, Gd, (((2,), (1,)), ((), ())))
    A1 = (Gd[None, None] * a2[:, :, None, :]).reshape(K * BB, P, EP)
    M2 = jax.lax.dot_general(A1, Gs, (((2,), (1,)), ((), ())))
    feat2b = feat2.reshape(K * BB, P, OUT)
    num2 = jax.lax.dot_general(M2, feat2b,
                               (((2,), (1,)), ((0,), (0,))))
    inv2 = jnp.where(den2 > 0, 1.0 / jnp.maximum(den2, 1e-30), 0.0)
    out2 = num2 * inv2.reshape(K * BB, P, 1)
    res2o = feat2w[:, :, OUT:2 * OUT]
    h2 = jnp.maximum(
        BN_SCALE * ((out2.reshape(K, BB, P, OUT)
                     + res2o.reshape(K, BB, P, OUT))
                    + b2[:, None, None, :]), 0.0)

    graph_rep = jnp.mean(h2, axis=2)

    h2r = jnp.concatenate([h2, jnp.ones((K, BB, P, 1), jnp.float32)],
                          axis=-1).reshape(K, BB * P, OUT + 1)
    d1 = jnp.maximum(
        BN_SCALE * jax.lax.dot_general(h2r, D1_ref[...],
                                       (((2,), (1,)), ((0,), (0,)))),
        0.0)
    rec = jax.lax.dot_general(d1, D2f_ref[...],
                              (((2,), (1,)), ((0,), (0,)))) + d2b[:, 0:1]
    xr = xf_ref[...]
    sq = (rec - xr) ** 2
    partial = jnp.sum(sq, axis=0, keepdims=True) * (1.0 / (P * B * K))

    @pl.when(pl.program_id(0) == 0)
    def _():
        loss_ref[...] = jnp.zeros((1, BB * P), jnp.float32)
    loss_ref[...] += partial

    g = jax.lax.dot_general(graph_rep, Wfg_ref[...],
                            (((2,), (1,)), ((0,), (0,))))
    gsum = jnp.sum(g, axis=0)
    ho = jnp.dot(orig_ref[...], Wfo_ref[...])
    hN_ref[...] = jnp.maximum(BN_SCALE * (gsum + ho + bf_ref[...][None, :]),
                              0.0)


def _count_body(ps_hbm, pd_hbm, z_hbm, out_hbm, ps_v, pd_v, flat_v, ones_v,
                c_sh):
    c = lax.axis_index("c")
    s = lax.axis_index("s")
    wid = s * 2 + c
    seg = 262144 // 16
    pltpu.sync_copy(z_hbm, c_sh.at[pl.ds(s * seg, seg)])
    plsc.subcore_barrier()
    base = wid * (E // 32)
    pltpu.sync_copy(ps_hbm.at[pl.ds(base, E // 32)], ps_v)
    pltpu.sync_copy(pd_hbm.at[pl.ds(base, E // 32)], pd_v)
    for j in range(8):
        ones_v[pl.ds(j * 16, 16)] = jnp.ones((16,), jnp.float32)
    for j in range(32):
        s16 = ps_v[pl.ds(j * 16, 16)]
        d16 = pd_v[pl.ds(j * 16, 16)]
        flat_v[j // 8, pl.ds((j % 8) * 16, 16)] = d16 * B + s16
    for r in range(4):
        pltpu.sync_copy(ones_v, c_sh.at[flat_v.at[r]], add=True)
    plsc.subcore_barrier()
    pltpu.sync_copy(c_sh.at[pl.ds(s * seg, seg)],
                    out_hbm.at[c, pl.ds(s * seg, seg)])


def _build_counts(psrc, pdst):
    mesh = plsc.VectorSubcoreMesh(core_axis_name="c", subcore_axis_name="s")
    k = functools.partial(
        pl.kernel, mesh=mesh,
        out_type=jax.ShapeDtypeStruct((2, B * B), jnp.float32),
        scratch_types=[
            pltpu.VMEM((E // 32,), jnp.int32),
            pltpu.VMEM((E // 32,), jnp.int32),
            pltpu.VMEM((4, 128), jnp.int32),
            pltpu.VMEM((128,), jnp.float32),
            pltpu.VMEM_SHARED((B * B,), jnp.float32),
        ],
    )(_count_body)
    return k(psrc, pdst, jnp.zeros((262144 // 16,), jnp.float32))


def _patient_body(cpair_ref, mrow_ref, mcol_ref, hN_ref,
                  Wc_ref, bc_ref, Wl1_ref, bl1_ref, Wl2_ref, bl2_ref,
                  out_ref):
    C = cpair_ref[0] + cpair_ref[1]

    degS = jnp.sum(C, axis=0, keepdims=True)
    degD = jnp.sum(C, axis=1, keepdims=True)
    ns = jax.lax.rsqrt(jnp.maximum(degS, 1.0))
    nd = jax.lax.rsqrt(jnp.maximum(degD, 1.0))
    A = C * mcol_ref[...] * (mrow_ref[...] * ns)

    hN = hN_ref[...]
    hcur = hN
    hsum = hN
    for i in range(3):
        agg = jnp.dot(A, hcur)
        rst = jnp.dot(agg * nd, Wc_ref[i]) + bc_ref[i, :][None, :]
        hcur = jnp.maximum(BN_SCALE * rst, 0.0) + hcur
        hsum = hsum + hcur
    hm = hsum * 0.25

    z = jnp.dot(hm, Wl1_ref[...]) + bl1_ref[...][None, :]
    mu = jnp.mean(z, axis=-1, keepdims=True)
    var = jnp.mean((z - mu) ** 2, axis=-1, keepdims=True)
    zn = jnp.maximum((z - mu) * jax.lax.rsqrt(var + LN_EPS), 0.0)
    out_ref[...] = jnp.dot(zn, Wl2_ref[...]) + bl2_ref[...][None, :]


def kernel(plane_feats, plane_edge_index, patient_edge_index,
           original_features, mask, W1, a1s, a1d, res1, b1, W2, a2s, a2d,
           res2, b2, D1, d1b, D2, d2b, Wf, bf, Wc, bc, Wl1, bl1, Wl2, bl2):
    f32 = jnp.float32
    xk = plane_feats.reshape(B, K, P).transpose(1, 0, 2)
    xflat = xk.reshape(K, B * P)
    ef = plane_edge_index.astype(f32)
    idx8 = jnp.zeros((8, 2 * EP), f32)
    idx8 = idx8.at[0:2].set(jnp.tile(ef, (1, 2)))
    idx8 = idx8.at[2, EP:].set(1.0)

    W1k = W1[:, 0, :]
    res1k = res1[:, 0, :]
    w1s = jnp.sum(W1k.reshape(K, HEADS, HID) * a1s, axis=-1)
    w1d = jnp.sum(W1k.reshape(K, HEADS, HID) * a1d, axis=-1)
    wA = jnp.repeat(w1s, EP, axis=1)
    wB = jnp.repeat(w1d, EP, axis=1)
    hsel = (jnp.arange(HEADS * HID) >= HID).astype(f32)
    Wg = jnp.stack([W1k * (1.0 - hsel), W1k * hsel, res1k, b1], axis=1)
    D1e = jnp.concatenate([D1, d1b[:, None, :]], axis=1)
    w2s = jnp.einsum('kfo,ko->kf', W2, a2s[:, 0])
    w2d = jnp.einsum('kfo,ko->kf', W2, a2d[:, 0])
    W2w = jnp.concatenate([W2, res2, w2s[:, :, None], w2d[:, :, None],
                           jnp.zeros((K, NH, 6), f32)], axis=2)
    D2f = D2[:, :, 0]
    d2b8 = jnp.broadcast_to(d2b, (K, 8)) if d2b.shape[1] == 1 else d2b
    Wfo = Wf[:ORIG]
    Wfg = Wf[ORIG:].reshape(K, OUT, NH)

    nblk = B // BB
    grid_spec = pl.GridSpec(
        grid=(nblk,),
        in_specs=[
            pl.BlockSpec((8, 2 * EP), lambda i: (0, 0)),
            pl.BlockSpec((K, BB, P), lambda i: (0, i, 0)),
            pl.BlockSpec((K, BB * P), lambda i: (0, i)),
            pl.BlockSpec((BB, ORIG), lambda i: (i, 0)),
            pl.BlockSpec((K, 4, NH), lambda i: (0, 0, 0)),
            pl.BlockSpec((K, 2 * EP), lambda i: (0, 0)),
            pl.BlockSpec((K, 2 * EP), lambda i: (0, 0)),
            pl.BlockSpec((K, NH, 2 * OUT + 8), lambda i: (0, 0, 0)),
            pl.BlockSpec((K, OUT), lambda i: (0, 0)),
            pl.BlockSpec((K, OUT + 1, NH), lambda i: (0, 0, 0)),
            pl.BlockSpec((K, NH), lambda i: (0, 0)),
            pl.BlockSpec((K, 8), lambda i: (0, 0)),
            pl.BlockSpec((ORIG, NH), lambda i: (0, 0)),
            pl.BlockSpec((K, OUT, NH), lambda i: (0, 0, 0)),
            pl.BlockSpec((NH,), lambda i: (0,)),
        ],
        out_specs=[
            pl.BlockSpec((BB, NH), lambda i: (i, 0)),
            pl.BlockSpec((1, BB * P), lambda i: (0, 0)),
        ],
    )
    hN, loss = _pc(
        _plane_body,
        grid_spec=grid_spec,
        out_shape=[jax.ShapeDtypeStruct((B, NH), f32),
                   jax.ShapeDtypeStruct((1, BB * P), f32)],
    )(idx8, xk, xflat, original_features, Wg, wA, wB, W2w, b2,
      D1e, D2f, d2b8, Wfo, Wfg, bf)

    cpair = _build_counts(patient_edge_index[0], patient_edge_index[1])
    cpair = cpair.reshape(2, B, B)
    maskf = mask.astype(f32)
    mrow = maskf.reshape(1, B)
    mcol = maskf.reshape(B, 1)

    logits = _pc(
        _patient_body,
        out_shape=jax.ShapeDtypeStruct((B, NCLS), f32),
    )(cpair, mrow, mcol, hN, Wc, bc, Wl1, bl1, Wl2, bl2)

    return logits, jnp.sum(loss)


# speedup vs baseline: 1.2137x; 1.0039x over previous
import functools
import jax
import jax.numpy as jnp
import numpy as np
from jax import lax
from jax.experimental import pallas as pl
from jax.experimental.pallas import tpu as pltpu
from jax.experimental.pallas import tpu_sc as plsc

B = 512; K = 24; P = 16; EP = 32; E = 16384
FIN = 1; HID = 64; HEADS = 2; OUT = 32; NH = 128; ORIG = 256; NCLS = 2
BN_SCALE = 1.0 / np.sqrt(1.0 + 1e-5)
LN_EPS = 1e-5
BB = 64

_pc = pl.pallas_call


def _leaky(x):
    return jnp.where(x >= 0, x, 0.2 * x)


def _plane_body(idx_ref, x_ref, xf_ref, orig_ref, Wg_ref,
                wA_ref, wB_ref, W2_ref, b2_ref,
                D1_ref, D2f_ref, d2b_ref, Wfo_ref, Wfg_ref, bf_ref,
                hN_ref, loss_ref):
    src = idx_ref[0:1, 0:EP]
    dst = idx_ref[1:2, 0:EP]
    pio = jax.lax.broadcasted_iota(jnp.int32, (P, EP), 0).astype(jnp.float32)
    Gs = (pio == src).astype(jnp.float32)
    Gd = (pio == dst).astype(jnp.float32)
    src2 = idx_ref[0:1, :]
    dst2 = idx_ref[1:2, :]
    hrow = idx_ref[2:3, :]
    rio = jax.lax.broadcasted_iota(jnp.int32, (2 * P, 2 * EP), 0)
    rp = jnp.remainder(rio, P).astype(jnp.float32)
    rh = (rio // P).astype(jnp.float32)
    Gs2 = (rp == src2).astype(jnp.float32) * (rh == hrow)
    Gd2 = (rp == dst2).astype(jnp.float32) * (rh == hrow)

    x = x_ref[...]
    b2 = b2_ref[...]
    d2b = d2b_ref[...]
    xs2 = jax.lax.dot_general(x, Gs2[0:P] + Gs2[P:2 * P],
                              (((2,), (0,)), ((), ())))
    xd2 = jax.lax.dot_general(x, Gd2[0:P] + Gd2[P:2 * P],
                              (((2,), (0,)), ((), ())))
    wA = wA_ref[...]
    wB = wB_ref[...]
    e12 = _leaky(xs2 * wA[:, None, :] + xd2 * wB[:, None, :])
    m = jnp.max(e12, axis=2, keepdims=True)
    a12 = jnp.exp(e12 - m)
    den12 = jax.lax.dot_general(a12, Gd2, (((2,), (1,)), ((), ())))
    num12 = jax.lax.dot_general(a12 * xs2, Gd2, (((2,), (1,)), ((), ())))
    s1p = jnp.where(den12 > 0, num12 / jnp.maximum(den12, 1e-30), 0.0)
    s1_0 = s1p[:, :, 0:P]
    s1_1 = s1p[:, :, P:2 * P]
    G = jnp.concatenate([s1_0[..., None], s1_1[..., None], x[..., None],
                         jnp.ones((K, BB, P, 1), jnp.float32)],
                        axis=-1).reshape(K, BB * P, 4)
    h1r = jnp.maximum(
        BN_SCALE * jax.lax.dot_general(G, Wg_ref[...],
                                       (((2,), (1,)), ((0,), (0,)))),
        0.0)

    feat2w = jax.lax.dot_general(h1r, W2_ref[...],
                                 (((2,), (1,)), ((0,), (0,))))
    feat2 = feat2w[:, :, 0:OUT]
    el2 = feat2w[:, :, 2 * OUT].reshape(K, BB, P)
    er2 = feat2w[:, :, 2 * OUT + 1].reshape(K, BB, P)
    e2 = _leaky(jax.lax.dot_general(el2, Gs, (((2,), (0,)), ((), ())))
                + jax.lax.dot_general(er2, Gd, (((2,), (0,)), ((), ()))))
    m2 = jnp.max(e2, axis=2, keepdims=True)
    a2 = jnp.exp(e2 - m2)
    den2 = jax.lax.dot_general(a2, Gd, (((2,), (1,)), ((), ())))
    A1 = (Gd[None, None] * a2[:, :, None, :]).reshape(K * BB, P, EP)
    M2 = jax.lax.dot_general(A1, Gs, (((2,), (1,)), ((), ())))
    feat2b = feat2.reshape(K * BB, P, OUT)
    num2 = jax.lax.dot_general(M2, feat2b,
                               (((2,), (1,)), ((0,), (0,))))
    inv2 = jnp.where(den2 > 0, 1.0 / jnp.maximum(den2, 1e-30), 0.0)
    out2 = num2 * inv2.reshape(K * BB, P, 1)
    res2o = feat2w[:, :, OUT:2 * OUT]
    h2 = jnp.maximum(
        BN_SCALE * ((out2.reshape(K, BB, P, OUT)
                     + res2o.reshape(K, BB, P, OUT))
                    + b2[:, None, None, :]), 0.0)

    graph_rep = jnp.mean(h2, axis=2)

    h2r = jnp.concatenate([h2, jnp.ones((K, BB, P, 1), jnp.float32)],
                          axis=-1).reshape(K, BB * P, OUT + 1)
    d1 = jnp.maximum(
        BN_SCALE * jax.lax.dot_general(h2r, D1_ref[...],
                                       (((2,), (1,)), ((0,), (0,)))),
        0.0)
    rec = jnp.sum(d1 * D2f_ref[...][:, None, :], axis=-1) + d2b[:, 0:1]
    xr = xf_ref[...]
    sq = (rec - xr) ** 2
    partial = jnp.sum(sq, axis=0, keepdims=True) * (1.0 / (P * B * K))

    @pl.when(pl.program_id(0) == 0)
    def _():
        loss_ref[...] = jnp.zeros((1, BB * P), jnp.float32)
    loss_ref[...] += partial

    g = jax.lax.dot_general(graph_rep, Wfg_ref[...],
                            (((2,), (1,)), ((0,), (0,))))
    gsum = jnp.sum(g, axis=0)
    ho = jnp.dot(orig_ref[...], Wfo_ref[...])
    hN_ref[...] = jnp.maximum(BN_SCALE * (gsum + ho + bf_ref[...][None, :]),
                              0.0)


def _count_body(ps_hbm, pd_hbm, z_hbm, out_hbm, ps_v, pd_v, flat_v, ones_v,
                c_sh):
    c = lax.axis_index("c")
    s = lax.axis_index("s")
    wid = s * 2 + c
    seg = 262144 // 16
    pltpu.sync_copy(z_hbm, c_sh.at[pl.ds(s * seg, seg)])
    plsc.subcore_barrier()
    base = wid * (E // 32)
    pltpu.sync_copy(ps_hbm.at[pl.ds(base, E // 32)], ps_v)
    pltpu.sync_copy(pd_hbm.at[pl.ds(base, E // 32)], pd_v)
    for j in range(8):
        ones_v[pl.ds(j * 16, 16)] = jnp.ones((16,), jnp.float32)
    for j in range(32):
        s16 = ps_v[pl.ds(j * 16, 16)]
        d16 = pd_v[pl.ds(j * 16, 16)]
        flat_v[j // 8, pl.ds((j % 8) * 16, 16)] = d16 * B + s16
    for r in range(4):
        pltpu.sync_copy(ones_v, c_sh.at[flat_v.at[r]], add=True)
    plsc.subcore_barrier()
    pltpu.sync_copy(c_sh.at[pl.ds(s * seg, seg)],
                    out_hbm.at[c, pl.ds(s * seg, seg)])


def _build_counts(psrc, pdst):
    mesh = plsc.VectorSubcoreMesh(core_axis_name="c", subcore_axis_name="s")
    k = functools.partial(
        pl.kernel, mesh=mesh,
        out_type=jax.ShapeDtypeStruct((2, B * B), jnp.float32),
        scratch_types=[
            pltpu.VMEM((E // 32,), jnp.int32),
            pltpu.VMEM((E // 32,), jnp.int32),
            pltpu.VMEM((4, 128), jnp.int32),
            pltpu.VMEM((128,), jnp.float32),
            pltpu.VMEM_SHARED((B * B,), jnp.float32),
        ],
    )(_count_body)
    return k(psrc, pdst, jnp.zeros((262144 // 16,), jnp.float32))


def _patient_body(cpair_ref, mrow_ref, mcol_ref, hN_ref,
                  Wc_ref, bc_ref, Wl1_ref, bl1_ref, Wl2_ref, bl2_ref,
                  out_ref):
    C = cpair_ref[0] + cpair_ref[1]

    degS = jnp.sum(C, axis=0, keepdims=True)
    degD = jnp.sum(C, axis=1, keepdims=True)
    ns = jax.lax.rsqrt(jnp.maximum(degS, 1.0))
    nd = jax.lax.rsqrt(jnp.maximum(degD, 1.0))
    A = C * mcol_ref[...] * (mrow_ref[...] * ns)

    hN = hN_ref[...]
    hcur = hN
    hsum = hN
    for i in range(3):
        agg = jnp.dot(A, hcur)
        rst = jnp.dot(agg * nd, Wc_ref[i]) + bc_ref[i, :][None, :]
        hcur = jnp.maximum(BN_SCALE * rst, 0.0) + hcur
        hsum = hsum + hcur
    hm = hsum * 0.25

    z = jnp.dot(hm, Wl1_ref[...]) + bl1_ref[...][None, :]
    mu = jnp.mean(z, axis=-1, keepdims=True)
    var = jnp.mean((z - mu) ** 2, axis=-1, keepdims=True)
    zn = jnp.maximum((z - mu) * jax.lax.rsqrt(var + LN_EPS), 0.0)
    out_ref[...] = jnp.dot(zn, Wl2_ref[...]) + bl2_ref[...][None, :]


def kernel(plane_feats, plane_edge_index, patient_edge_index,
           original_features, mask, W1, a1s, a1d, res1, b1, W2, a2s, a2d,
           res2, b2, D1, d1b, D2, d2b, Wf, bf, Wc, bc, Wl1, bl1, Wl2, bl2):
    f32 = jnp.float32
    xk = plane_feats.reshape(B, K, P).transpose(1, 0, 2)
    xflat = xk.reshape(K, B * P)
    ef = plane_edge_index.astype(f32)
    idx8 = jnp.zeros((8, 2 * EP), f32)
    idx8 = idx8.at[0:2].set(jnp.tile(ef, (1, 2)))
    idx8 = idx8.at[2, EP:].set(1.0)

    W1k = W1[:, 0, :]
    res1k = res1[:, 0, :]
    w1s = jnp.sum(W1k.reshape(K, HEADS, HID) * a1s, axis=-1)
    w1d = jnp.sum(W1k.reshape(K, HEADS, HID) * a1d, axis=-1)
    wA = jnp.repeat(w1s, EP, axis=1)
    wB = jnp.repeat(w1d, EP, axis=1)
    hsel = (jnp.arange(HEADS * HID) >= HID).astype(f32)
    Wg = jnp.stack([W1k * (1.0 - hsel), W1k * hsel, res1k, b1], axis=1)
    D1e = jnp.concatenate([D1, d1b[:, None, :]], axis=1)
    w2s = jnp.einsum('kfo,ko->kf', W2, a2s[:, 0])
    w2d = jnp.einsum('kfo,ko->kf', W2, a2d[:, 0])
    W2w = jnp.concatenate([W2, res2, w2s[:, :, None], w2d[:, :, None],
                           jnp.zeros((K, NH, 6), f32)], axis=2)
    D2f = D2[:, :, 0]
    d2b8 = jnp.broadcast_to(d2b, (K, 8)) if d2b.shape[1] == 1 else d2b
    Wfo = Wf[:ORIG]
    Wfg = Wf[ORIG:].reshape(K, OUT, NH)

    nblk = B // BB
    grid_spec = pl.GridSpec(
        grid=(nblk,),
        in_specs=[
            pl.BlockSpec((8, 2 * EP), lambda i: (0, 0)),
            pl.BlockSpec((K, BB, P), lambda i: (0, i, 0)),
            pl.BlockSpec((K, BB * P), lambda i: (0, i)),
            pl.BlockSpec((BB, ORIG), lambda i: (i, 0)),
            pl.BlockSpec((K, 4, NH), lambda i: (0, 0, 0)),
            pl.BlockSpec((K, 2 * EP), lambda i: (0, 0)),
            pl.BlockSpec((K, 2 * EP), lambda i: (0, 0)),
            pl.BlockSpec((K, NH, 2 * OUT + 8), lambda i: (0, 0, 0)),
            pl.BlockSpec((K, OUT), lambda i: (0, 0)),
            pl.BlockSpec((K, OUT + 1, NH), lambda i: (0, 0, 0)),
            pl.BlockSpec((K, NH), lambda i: (0, 0)),
            pl.BlockSpec((K, 8), lambda i: (0, 0)),
            pl.BlockSpec((ORIG, NH), lambda i: (0, 0)),
            pl.BlockSpec((K, OUT, NH), lambda i: (0, 0, 0)),
            pl.BlockSpec((NH,), lambda i: (0,)),
        ],
        out_specs=[
            pl.BlockSpec((BB, NH), lambda i: (i, 0)),
            pl.BlockSpec((1, BB * P), lambda i: (0, 0)),
        ],
    )
    hN, loss = _pc(
        _plane_body,
        grid_spec=grid_spec,
        out_shape=[jax.ShapeDtypeStruct((B, NH), f32),
                   jax.ShapeDtypeStruct((1, BB * P), f32)],
    )(idx8, xk, xflat, original_features, Wg, wA, wB, W2w, b2,
      D1e, D2f, d2b8, Wfo, Wfg, bf)

    cpair = _build_counts(patient_edge_index[0], patient_edge_index[1])
    cpair = cpair.reshape(2, B, B)
    maskf = mask.astype(f32)
    mrow = maskf.reshape(1, B)
    mcol = maskf.reshape(B, 1)

    logits = _pc(
        _patient_body,
        out_shape=jax.ShapeDtypeStruct((B, NCLS), f32),
    )(cpair, mrow, mcol, hN, Wc, bc, Wl1, bl1, Wl2, bl2)

    return logits, jnp.sum(loss)


# speedup vs baseline: 1.3827x; 1.1392x over previous
import functools
import jax
import jax.numpy as jnp
import numpy as np
from jax import lax
from jax.experimental import pallas as pl
from jax.experimental.pallas import tpu as pltpu
from jax.experimental.pallas import tpu_sc as plsc

B = 512; K = 24; P = 16; EP = 32; E = 16384
FIN = 1; HID = 64; HEADS = 2; OUT = 32; NH = 128; ORIG = 256; NCLS = 2
BN_SCALE = 1.0 / np.sqrt(1.0 + 1e-5)
LN_EPS = 1e-5
BB = 64

_pc = pl.pallas_call


def _leaky(x):
    return jnp.where(x >= 0, x, 0.2 * x)


def _plane_body(idx_ref, x_ref, xf_ref, orig_ref, Wg_ref,
                wA_ref, wB_ref, W2_ref, b2_ref,
                D1_ref, D2f_ref, d2b_ref, Wfo_ref, Wfg_ref, bf_ref,
                hN_ref, loss_ref):
    src = idx_ref[0:1, 0:EP]
    dst = idx_ref[1:2, 0:EP]
    pio = jax.lax.broadcasted_iota(jnp.int32, (P, EP), 0).astype(jnp.float32)
    Gs = (pio == src).astype(jnp.float32)
    Gd = (pio == dst).astype(jnp.float32)
    src2 = idx_ref[0:1, :]
    dst2 = idx_ref[1:2, :]
    hrow = idx_ref[2:3, :]
    rio = jax.lax.broadcasted_iota(jnp.int32, (2 * P, 2 * EP), 0)
    rp = jnp.remainder(rio, P).astype(jnp.float32)
    rh = (rio // P).astype(jnp.float32)
    Gs2 = (rp == src2).astype(jnp.float32) * (rh == hrow)
    Gd2 = (rp == dst2).astype(jnp.float32) * (rh == hrow)

    x = x_ref[...]
    b2 = b2_ref[...]
    d2b = d2b_ref[...]
    xs2 = jax.lax.dot_general(x, Gs2[0:P] + Gs2[P:2 * P],
                              (((2,), (0,)), ((), ())))
    xd2 = jax.lax.dot_general(x, Gd2[0:P] + Gd2[P:2 * P],
                              (((2,), (0,)), ((), ())))
    wA = wA_ref[...]
    wB = wB_ref[...]
    e12 = _leaky(xs2 * wA[:, None, :] + xd2 * wB[:, None, :])
    m = jnp.max(e12, axis=2, keepdims=True)
    a12 = jnp.exp(e12 - m)
    den12 = jax.lax.dot_general(a12, Gd2, (((2,), (1,)), ((), ())))
    num12 = jax.lax.dot_general(a12 * xs2, Gd2, (((2,), (1,)), ((), ())))
    s1p = jnp.where(den12 > 0, num12 / jnp.maximum(den12, 1e-30), 0.0)
    s1_0 = s1p[:, :, 0:P]
    s1_1 = s1p[:, :, P:2 * P]
    U = jnp.concatenate([s1p, x, jnp.ones((K, BB, P), jnp.float32)],
                        axis=2)
    G = jnp.swapaxes(U.reshape(K, BB, 4, P), 2, 3).reshape(K, BB * P, 4)
    h1r = jnp.maximum(
        BN_SCALE * jax.lax.dot_general(G, Wg_ref[...],
                                       (((2,), (1,)), ((0,), (0,)))),
        0.0)

    feat2w = jax.lax.dot_general(h1r, W2_ref[...],
                                 (((2,), (1,)), ((0,), (0,))))
    feat2 = feat2w[:, :, 0:OUT]
    el2 = feat2w[:, :, 2 * OUT].reshape(K, BB, P)
    er2 = feat2w[:, :, 2 * OUT + 1].reshape(K, BB, P)
    e2 = _leaky(jax.lax.dot_general(el2, Gs, (((2,), (0,)), ((), ())))
                + jax.lax.dot_general(er2, Gd, (((2,), (0,)), ((), ()))))
    m2 = jnp.max(e2, axis=2, keepdims=True)
    a2 = jnp.exp(e2 - m2)
    den2 = jax.lax.dot_general(a2, Gd, (((2,), (1,)), ((), ())))
    A1 = (Gd[None, None] * a2[:, :, None, :]).reshape(K * BB, P, EP)
    M2 = jax.lax.dot_general(A1, Gs, (((2,), (1,)), ((), ())))
    feat2b = feat2.reshape(K * BB, P, OUT)
    num2 = jax.lax.dot_general(M2, feat2b,
                               (((2,), (1,)), ((0,), (0,))))
    inv2 = jnp.where(den2 > 0, 1.0 / jnp.maximum(den2, 1e-30), 0.0)
    out2 = num2 * inv2.reshape(K * BB, P, 1)
    res2o = feat2w[:, :, OUT:2 * OUT]
    h2 = jnp.maximum(
        BN_SCALE * ((out2.reshape(K, BB, P, OUT)
                     + res2o.reshape(K, BB, P, OUT))
                    + b2[:, None, None, :]), 0.0)

    graph_rep = jnp.mean(h2, axis=2)

    h2r = jnp.concatenate([h2, jnp.ones((K, BB, P, 1), jnp.float32)],
                          axis=-1).reshape(K, BB * P, OUT + 1)
    d1 = jnp.maximum(
        BN_SCALE * jax.lax.dot_general(h2r, D1_ref[...],
                                       (((2,), (1,)), ((0,), (0,)))),
        0.0)
    rec = jnp.sum(d1 * D2f_ref[...][:, None, :], axis=-1) + d2b[:, 0:1]
    xr = xf_ref[...]
    sq = (rec - xr) ** 2
    partial = jnp.sum(sq, axis=0, keepdims=True) * (1.0 / (P * B * K))

    @pl.when(pl.program_id(0) == 0)
    def _():
        loss_ref[...] = jnp.zeros((1, BB * P), jnp.float32)
    loss_ref[...] += partial

    g = jax.lax.dot_general(graph_rep, Wfg_ref[...],
                            (((2,), (1,)), ((0,), (0,))))
    gsum = jnp.sum(g, axis=0)
    ho = jnp.dot(orig_ref[...], Wfo_ref[...])
    hN_ref[...] = jnp.maximum(BN_SCALE * (gsum + ho + bf_ref[...][None, :]),
                              0.0)


def _count_body(ps_hbm, pd_hbm, z_hbm, out_hbm, ps_v, pd_v, flat_v, ones_v,
                c_sh):
    c = lax.axis_index("c")
    s = lax.axis_index("s")
    wid = s * 2 + c
    seg = 262144 // 16
    pltpu.sync_copy(z_hbm, c_sh.at[pl.ds(s * seg, seg)])
    plsc.subcore_barrier()
    base = wid * (E // 32)
    pltpu.sync_copy(ps_hbm.at[pl.ds(base, E // 32)], ps_v)
    pltpu.sync_copy(pd_hbm.at[pl.ds(base, E // 32)], pd_v)
    for j in range(8):
        ones_v[pl.ds(j * 16, 16)] = jnp.ones((16,), jnp.float32)
    for j in range(32):
        s16 = ps_v[pl.ds(j * 16, 16)]
        d16 = pd_v[pl.ds(j * 16, 16)]
        flat_v[j // 8, pl.ds((j % 8) * 16, 16)] = d16 * B + s16
    for r in range(4):
        pltpu.sync_copy(ones_v, c_sh.at[flat_v.at[r]], add=True)
    plsc.subcore_barrier()
    pltpu.sync_copy(c_sh.at[pl.ds(s * seg, seg)],
                    out_hbm.at[c, pl.ds(s * seg, seg)])


def _build_counts(psrc, pdst):
    mesh = plsc.VectorSubcoreMesh(core_axis_name="c", subcore_axis_name="s")
    k = functools.partial(
        pl.kernel, mesh=mesh,
        out_type=jax.ShapeDtypeStruct((2, B * B), jnp.float32),
        scratch_types=[
            pltpu.VMEM((E // 32,), jnp.int32),
            pltpu.VMEM((E // 32,), jnp.int32),
            pltpu.VMEM((4, 128), jnp.int32),
            pltpu.VMEM((128,), jnp.float32),
            pltpu.VMEM_SHARED((B * B,), jnp.float32),
        ],
    )(_count_body)
    return k(psrc, pdst, jnp.zeros((262144 // 16,), jnp.float32))


def _patient_body(cpair_ref, mrow_ref, mcol_ref, hN_ref,
                  Wc_ref, bc_ref, Wl1_ref, bl1_ref, Wl2_ref, bl2_ref,
                  out_ref):
    C = cpair_ref[0] + cpair_ref[1]

    degS = jnp.sum(C, axis=0, keepdims=True)
    degD = jnp.sum(C, axis=1, keepdims=True)
    ns = jax.lax.rsqrt(jnp.maximum(degS, 1.0))
    nd = jax.lax.rsqrt(jnp.maximum(degD, 1.0))
    A = C * mcol_ref[...] * (mrow_ref[...] * ns)

    hN = hN_ref[...]
    hcur = hN
    hsum = hN
    for i in range(3):
        agg = jnp.dot(A, hcur)
        rst = jnp.dot(agg * nd, Wc_ref[i]) + bc_ref[i, :][None, :]
        hcur = jnp.maximum(BN_SCALE * rst, 0.0) + hcur
        hsum = hsum + hcur
    hm = hsum * 0.25

    z = jnp.dot(hm, Wl1_ref[...]) + bl1_ref[...][None, :]
    mu = jnp.mean(z, axis=-1, keepdims=True)
    var = jnp.mean((z - mu) ** 2, axis=-1, keepdims=True)
    zn = jnp.maximum((z - mu) * jax.lax.rsqrt(var + LN_EPS), 0.0)
    out_ref[...] = jnp.dot(zn, Wl2_ref[...]) + bl2_ref[...][None, :]


def kernel(plane_feats, plane_edge_index, patient_edge_index,
           original_features, mask, W1, a1s, a1d, res1, b1, W2, a2s, a2d,
           res2, b2, D1, d1b, D2, d2b, Wf, bf, Wc, bc, Wl1, bl1, Wl2, bl2):
    f32 = jnp.float32
    xk = plane_feats.reshape(B, K, P).transpose(1, 0, 2)
    xflat = xk.reshape(K, B * P)
    ef = plane_edge_index.astype(f32)
    idx8 = jnp.zeros((8, 2 * EP), f32)
    idx8 = idx8.at[0:2].set(jnp.tile(ef, (1, 2)))
    idx8 = idx8.at[2, EP:].set(1.0)

    W1k = W1[:, 0, :]
    res1k = res1[:, 0, :]
    w1s = jnp.sum(W1k.reshape(K, HEADS, HID) * a1s, axis=-1)
    w1d = jnp.sum(W1k.reshape(K, HEADS, HID) * a1d, axis=-1)
    wA = jnp.repeat(w1s, EP, axis=1)
    wB = jnp.repeat(w1d, EP, axis=1)
    hsel = (jnp.arange(HEADS * HID) >= HID).astype(f32)
    Wg = jnp.stack([W1k * (1.0 - hsel), W1k * hsel, res1k, b1], axis=1)
    D1e = jnp.concatenate([D1, d1b[:, None, :]], axis=1)
    w2s = jnp.einsum('kfo,ko->kf', W2, a2s[:, 0])
    w2d = jnp.einsum('kfo,ko->kf', W2, a2d[:, 0])
    W2w = jnp.concatenate([W2, res2, w2s[:, :, None], w2d[:, :, None],
                           jnp.zeros((K, NH, 6), f32)], axis=2)
    D2f = D2[:, :, 0]
    d2b8 = jnp.broadcast_to(d2b, (K, 8)) if d2b.shape[1] == 1 else d2b
    Wfo = Wf[:ORIG]
    Wfg = Wf[ORIG:].reshape(K, OUT, NH)

    nblk = B // BB
    grid_spec = pl.GridSpec(
        grid=(nblk,),
        in_specs=[
            pl.BlockSpec((8, 2 * EP), lambda i: (0, 0)),
            pl.BlockSpec((K, BB, P), lambda i: (0, i, 0)),
            pl.BlockSpec((K, BB * P), lambda i: (0, i)),
            pl.BlockSpec((BB, ORIG), lambda i: (i, 0)),
            pl.BlockSpec((K, 4, NH), lambda i: (0, 0, 0)),
            pl.BlockSpec((K, 2 * EP), lambda i: (0, 0)),
            pl.BlockSpec((K, 2 * EP), lambda i: (0, 0)),
            pl.BlockSpec((K, NH, 2 * OUT + 8), lambda i: (0, 0, 0)),
            pl.BlockSpec((K, OUT), lambda i: (0, 0)),
            pl.BlockSpec((K, OUT + 1, NH), lambda i: (0, 0, 0)),
            pl.BlockSpec((K, NH), lambda i: (0, 0)),
            pl.BlockSpec((K, 8), lambda i: (0, 0)),
            pl.BlockSpec((ORIG, NH), lambda i: (0, 0)),
            pl.BlockSpec((K, OUT, NH), lambda i: (0, 0, 0)),
            pl.BlockSpec((NH,), lambda i: (0,)),
        ],
        out_specs=[
            pl.BlockSpec((BB, NH), lambda i: (i, 0)),
            pl.BlockSpec((1, BB * P), lambda i: (0, 0)),
        ],
    )
    hN, loss = _pc(
        _plane_body,
        grid_spec=grid_spec,
        out_shape=[jax.ShapeDtypeStruct((B, NH), f32),
                   jax.ShapeDtypeStruct((1, BB * P), f32)],
    )(idx8, xk, xflat, original_features, Wg, wA, wB, W2w, b2,
      D1e, D2f, d2b8, Wfo, Wfg, bf)

    cpair = _build_counts(patient_edge_index[0], patient_edge_index[1])
    cpair = cpair.reshape(2, B, B)
    maskf = mask.astype(f32)
    mrow = maskf.reshape(1, B)
    mcol = maskf.reshape(B, 1)

    logits = _pc(
        _patient_body,
        out_shape=jax.ShapeDtypeStruct((B, NCLS), f32),
    )(cpair, mrow, mcol, hN, Wc, bc, Wl1, bl1, Wl2, bl2)

    return logits, jnp.sum(loss)
